# Initial kernel scaffold; baseline (speedup 1.0000x reference)
import jax
import jax.numpy as jnp
from jax.experimental import pallas as pl


def kernel(x, W_embed, b_embed, Wm0, bm0, Wu0, bu0, Wm1, bm1, Wu1, bu1, W_out, b_out, edge_index):
    raise NotImplementedError("write your pallas kernel here")


# speedup vs baseline: 1.1536x; 1.1536x over previous
import functools

import jax
import jax.numpy as jnp
from jax import lax
from jax.experimental import pallas as pl
from jax.experimental.pallas import tpu as pltpu
from jax.experimental.pallas import tpu_sc as plsc

N = 50000
E = 800000
ATOM = 10
H = 128
LAT = 128

NC = 2
NS = 16
NW = NC * NS

C = 64
NCHUNK = E // C
PASS_LO = (0, 12504, 25008, 37512)
PASS_SZ = (12504, 12504, 12504, 12488)
TRASH = 12504
ACC_ROWS = 12544
ZROWS = ACC_ROWS // NS
WROWS = 776

BN = 1000
NBLK = N // BN


def _relu(v):
    return jnp.maximum(v, 0.0)


def _dot(a, b):
    return jnp.dot(a, b, preferred_element_type=jnp.float32)


def _embed_body(x_ref, We, be, WA, WB, bm, h_ref, hs_ref, hd_ref):
    h = _dot(x_ref[...], We[...]) + be[...]
    h_ref[...] = h
    hs_ref[...] = _dot(h, WA[...])
    hd_ref[...] = _dot(h, WB[...]) + bm[...]


def _upd_body(h_ref, a0, a1, Wu, bu, WA, WB, bm, h_out, hs_ref, hd_ref):
    agg = a0[...] + a1[...]
    h = _relu(h_ref[...] + _dot(agg, Wu[...]) + bu[...])
    h_out[...] = h
    hs_ref[...] = _dot(h, WA[...])
    hd_ref[...] = _dot(h, WB[...]) + bm[...]


def _fin_body(h_ref, a0, a1, Wu, bu, Wo, bo, z_ref):
    agg = a0[...] + a1[...]
    h = _relu(h_ref[...] + _dot(agg, Wu[...]) + bu[...])
    z_ref[...] = _dot(h, Wo[...]) + bo[...]


def _row_spec(off=0):
    return pl.BlockSpec((BN, H), lambda i, o=off: (i + o, 0))


def _full(shape):
    return pl.BlockSpec(shape, lambda i: (0,) * len(shape))


_OUT3 = [jax.ShapeDtypeStruct((N, H), jnp.float32)] * 3

_embed_call = pl.pallas_call(
    _embed_body,
    grid=(NBLK,),
    in_specs=[
        pl.BlockSpec((BN, ATOM), lambda i: (i, 0)),
        _full((ATOM, H)), _full((1, H)), _full((H, H)), _full((H, H)),
        _full((1, H)),
    ],
    out_specs=[_row_spec()] * 3,
    out_shape=_OUT3,
)

_upd_call = pl.pallas_call(
    _upd_body,
    grid=(NBLK,),
    in_specs=[
        _row_spec(), _row_spec(), _row_spec(NBLK),
        _full((H, H)), _full((1, H)), _full((H, H)), _full((H, H)),
        _full((1, H)),
    ],
    out_specs=[_row_spec()] * 3,
    out_shape=_OUT3,
)

_fin_call = pl.pallas_call(
    _fin_body,
    grid=(NBLK,),
    in_specs=[
        _row_spec(), _row_spec(), _row_spec(NBLK),
        _full((H, H)), _full((1, H)), _full((H, LAT)), _full((1, LAT)),
    ],
    out_specs=_row_spec(),
    out_shape=jax.ShapeDtypeStruct((N, LAT), jnp.float32),
)


_sc_mesh = plsc.VectorSubcoreMesh(core_axis_name="c", subcore_axis_name="s")


@functools.partial(
    pl.kernel,
    out_type=jax.ShapeDtypeStruct((NC * N, H), jnp.float32),
    mesh=_sc_mesh,
    scratch_types=[
        pltpu.VMEM((C,), jnp.int32),
        pltpu.VMEM((C,), jnp.int32),
        pltpu.VMEM((C,), jnp.int32),
        pltpu.VMEM((C, H), jnp.float32),
        pltpu.VMEM((C, H), jnp.float32),
        pltpu.VMEM_SHARED((ACC_ROWS, H), jnp.float32),
        pltpu.SemaphoreType.DMA,
        pltpu.SemaphoreType.DMA,
    ],
)
def _edge_kernel(hs_hbm, hd_hbm, src_hbm, dst_hbm, agg_hbm,
                 sidx, didx, lidx, bufA, bufB, acc, semA, semB):
    c = lax.axis_index("c")
    s = lax.axis_index("s")
    wid = c * NS + s
    ntrip = jnp.where(wid < NCHUNK - (NCHUNK // NW) * NW,
                      NCHUNK // NW + 1, NCHUNK // NW)

    for p in range(len(PASS_LO)):
        lo = PASS_LO[p]
        psz = PASS_SZ[p]

        def _zero_buf(r, carry):
            for k in range(H // 16):
                bufA[r, pl.ds(k * 16, 16)] = jnp.zeros((16,), jnp.float32)
            return carry
        lax.fori_loop(0, C, _zero_buf, 0)
        zbase = s * ZROWS
        for j in range(ZROWS // C):
            pltpu.sync_copy(bufA, acc.at[pl.ds(zbase + j * C, C)])
        rem = ZROWS - (ZROWS // C) * C
        if rem:
            pltpu.sync_copy(bufA.at[pl.ds(0, rem)],
                            acc.at[pl.ds(zbase + (ZROWS // C) * C, rem)])
        plsc.subcore_barrier()

        def _chunk(i, carry):
            eb = (wid + i * NW) * C
            pltpu.sync_copy(src_hbm.at[pl.ds(eb, C)], sidx)
            pltpu.sync_copy(dst_hbm.at[pl.ds(eb, C)], didx)
            for k in range(C // 16):
                d = didx[pl.ds(k * 16, 16)]
                l = d - lo
                ok = (l >= 0) & (l < psz)
                lidx[pl.ds(k * 16, 16)] = jnp.where(ok, l, TRASH)
            ga = pltpu.async_copy(hs_hbm.at[sidx], bufA, semA)
            gb = pltpu.async_copy(hd_hbm.at[didx], bufB, semB)
            ga.wait()
            gb.wait()

            def _rows(r, rc):
                for k in range(H // 16):
                    a = bufA[r, pl.ds(k * 16, 16)]
                    b = bufB[r, pl.ds(k * 16, 16)]
                    bufA[r, pl.ds(k * 16, 16)] = _relu(a + b)
                return rc
            lax.fori_loop(0, C, _rows, 0)
            pltpu.sync_copy(bufA, acc.at[lidx], add=True)
            return carry
        lax.fori_loop(0, ntrip, _chunk, 0)
        plsc.subcore_barrier()

        obase = c * N + lo
        wbase = s * WROWS
        pltpu.sync_copy(acc.at[pl.ds(wbase, WROWS)],
                        agg_hbm.at[pl.ds(obase + wbase, WROWS)])

        @pl.when(s == 0)
        def _tail():
            pltpu.sync_copy(acc.at[pl.ds(NS * WROWS, psz - NS * WROWS)],
                            agg_hbm.at[pl.ds(obase + NS * WROWS,
                                             psz - NS * WROWS)])
        plsc.subcore_barrier()


def kernel(x, W_embed, b_embed, Wm0, bm0, Wu0, bu0, Wm1, bm1, Wu1, bu1,
           W_out, b_out, edge_index):
    src = edge_index[0]
    dst = edge_index[1]
    be = b_embed.reshape(1, H)
    bm0r = bm0.reshape(1, H)
    bm1r = bm1.reshape(1, H)
    bu0r = bu0.reshape(1, H)
    bu1r = bu1.reshape(1, H)
    bor = b_out.reshape(1, LAT)

    h0, hs0, hd0 = _embed_call(x, W_embed, be, Wm0[:H], Wm0[H:], bm0r)
    agg0 = _edge_kernel(hs0, hd0, src, dst)
    h1, hs1, hd1 = _upd_call(h0, agg0, agg0, Wu0, bu0r, Wm1[:H], Wm1[H:],
                             bm1r)
    agg1 = _edge_kernel(hs1, hd1, src, dst)
    z = _fin_call(h1, agg1, agg1, Wu1, bu1r, W_out, bor)
    return z


# speedup vs baseline: 4.7271x; 4.0977x over previous
import functools

import jax
import jax.numpy as jnp
from jax import lax
from jax.experimental import pallas as pl
from jax.experimental.pallas import tpu as pltpu
from jax.experimental.pallas import tpu_sc as plsc

N = 50000
E = 800000
ATOM = 10
H = 128
LAT = 128

NC = 2
NS = 16
NW = NC * NS

C = 128
NCHUNK = E // C
NBUF = 3
CP = 4
CW = H // CP

BN = 1024
NBLK = 49
PAD = NBLK * BN
ZROWS = PAD // NS


def _relu(v):
    return jnp.maximum(v, 0.0)


def _dot(a, b):
    return jnp.dot(a, b, preferred_element_type=jnp.float32)


def _embed_body(x_ref, We, be, WA, WB, bm, h_ref, hs_ref, hd_ref):
    h = _dot(x_ref[...], We[...]) + be[...]
    h_ref[...] = h
    hs_ref[...] = _dot(h, WA[...])
    hd_ref[...] = _dot(h, WB[...]) + bm[...]


def _upd_body(h_ref, p_ref, Wu, bu, WA, WB, bm, h_out, hs_ref, hd_ref):
    agg = p_ref[0] + p_ref[1]
    h = _relu(h_ref[...] + _dot(agg, Wu[...]) + bu[...])
    h_out[...] = h
    hs_ref[...] = _dot(h, WA[...])
    hd_ref[...] = _dot(h, WB[...]) + bm[...]


def _fin_body(h_ref, p_ref, Wu, bu, Wo, bo, z_ref):
    agg = p_ref[0] + p_ref[1]
    h = _relu(h_ref[...] + _dot(agg, Wu[...]) + bu[...])
    z_ref[...] = _dot(h, Wo[...]) + bo[...]


def _row_spec():
    return pl.BlockSpec((BN, H), lambda i: (i, 0))


def _part_spec():
    return pl.BlockSpec((2, BN, H), lambda i: (0, i, 0))


def _full(shape):
    return pl.BlockSpec(shape, lambda i: (0,) * len(shape))


_OUTH = [jax.ShapeDtypeStruct((N, H), jnp.float32),
         jax.ShapeDtypeStruct((PAD, H), jnp.float32),
         jax.ShapeDtypeStruct((PAD, H), jnp.float32)]

_embed_call = pl.pallas_call(
    _embed_body,
    grid=(NBLK,),
    in_specs=[
        pl.BlockSpec((BN, ATOM), lambda i: (i, 0)),
        _full((ATOM, H)), _full((1, H)), _full((H, H)), _full((H, H)),
        _full((1, H)),
    ],
    out_specs=[_row_spec()] * 3,
    out_shape=_OUTH,
)

_upd_call = pl.pallas_call(
    _upd_body,
    grid=(NBLK,),
    in_specs=[
        _row_spec(), _part_spec(),
        _full((H, H)), _full((1, H)), _full((H, H)), _full((H, H)),
        _full((1, H)),
    ],
    out_specs=[_row_spec()] * 3,
    out_shape=_OUTH,
)

_fin_call = pl.pallas_call(
    _fin_body,
    grid=(NBLK,),
    in_specs=[
        _row_spec(), _part_spec(),
        _full((H, H)), _full((1, H)), _full((H, LAT)), _full((1, LAT)),
    ],
    out_specs=_row_spec(),
    out_shape=jax.ShapeDtypeStruct((N, LAT), jnp.float32),
)


_sc_mesh = plsc.VectorSubcoreMesh(core_axis_name="c", subcore_axis_name="s")


@functools.partial(
    pl.kernel,
    out_type=jax.ShapeDtypeStruct((2, PAD, CP, CW), jnp.float32),
    mesh=_sc_mesh,
    compiler_params=pltpu.CompilerParams(use_tc_tiling_on_sc=False),
    scratch_types=[
        [pltpu.VMEM((C,), jnp.int32) for _ in range(NBUF)],
        [pltpu.VMEM((C,), jnp.int32) for _ in range(NBUF)],
        [pltpu.VMEM((C,), jnp.int32) for _ in range(NBUF)],
        [pltpu.VMEM((C, CW), jnp.float32) for _ in range(NBUF)],
        [pltpu.VMEM((C, CW), jnp.float32) for _ in range(NBUF)],
        pltpu.VMEM_SHARED((PAD, CW), jnp.float32),
        [pltpu.SemaphoreType.DMA for _ in range(NBUF)],
        [pltpu.SemaphoreType.DMA for _ in range(NBUF)],
        [pltpu.SemaphoreType.DMA for _ in range(NBUF)],
        [pltpu.SemaphoreType.DMA for _ in range(NBUF)],
    ],
)
def _edge_kernel(hs_hbm, hd_hbm, src_hbm, dst_hbm, agg_hbm,
                 sidx, didx, dgix, bufA, bufB, acc, semI, semA, semB, semS):
    c = lax.axis_index("c")
    s = lax.axis_index("s")
    wid = c * NS + s
    base_trips = NCHUNK // NW
    ntrip = jnp.where(wid < NCHUNK - base_trips * NW, base_trips + 1,
                      base_trips)

    for p in range(CP):
        def _zero_buf(r, carry):
            for k in range(CW // 16):
                bufA[0][r, pl.ds(k * 16, 16)] = jnp.zeros((16,), jnp.float32)
            return carry
        lax.fori_loop(0, C, _zero_buf, 0)
        zbase = s * ZROWS
        for t in range(ZROWS // C):
            pltpu.sync_copy(bufA[0], acc.at[pl.ds(zbase + t * C, C)])
        rem = ZROWS - (ZROWS // C) * C
        if rem:
            pltpu.sync_copy(bufA[0].at[pl.ds(0, rem)],
                            acc.at[pl.ds(zbase + (ZROWS // C) * C, rem)])
        plsc.subcore_barrier()

        def _slots(g, carry, p=p):
            for bb in range(NBUF):
                j = g * NBUF + bb
                bY = (bb - 1) % NBUF
                bZ = (bb - 2) % NBUF

                @pl.when((j >= NBUF) & (j - NBUF < ntrip))
                def _drain(bb=bb):
                    pltpu.make_async_copy(hs_hbm.at[pl.ds(0, C)], bufA[bb],
                                          semS[bb]).wait()

                @pl.when(j < ntrip)
                def _fire_idx(j=j, bb=bb):
                    eb = (wid + j * NW) * C
                    pltpu.async_copy(src_hbm.at[pl.ds(eb, C)], sidx[bb],
                                     semI[bb])
                    pltpu.async_copy(dst_hbm.at[pl.ds(eb, C)], didx[bb],
                                     semI[bb])

                @pl.when((j >= 1) & (j - 1 < ntrip))
                def _fire_gather(bY=bY, p=p):
                    pltpu.make_async_copy(src_hbm.at[pl.ds(0, C)],
                                          sidx[bY], semI[bY]).wait()
                    pltpu.make_async_copy(src_hbm.at[pl.ds(0, C)],
                                          didx[bY], semI[bY]).wait()
                    for k in range(C // 16):
                        sl = pl.ds(k * 16, 16)
                        sidx[bY][sl] = sidx[bY][sl] * CP + p
                        dgix[bY][sl] = didx[bY][sl] * CP + p
                    pltpu.async_copy(hs_hbm.at[sidx[bY]], bufA[bY], semA[bY])
                    pltpu.async_copy(hd_hbm.at[dgix[bY]], bufB[bY], semB[bY])

                @pl.when((j >= 2) & (j - 2 < ntrip))
                def _compute(bZ=bZ):
                    pltpu.make_async_copy(hs_hbm.at[pl.ds(0, C)], bufA[bZ],
                                          semA[bZ]).wait()
                    pltpu.make_async_copy(hs_hbm.at[pl.ds(0, C)], bufB[bZ],
                                          semB[bZ]).wait()

                    def _rows(rw, rc):
                        for k in range(CW // 16):
                            a = bufA[bZ][rw, pl.ds(k * 16, 16)]
                            b = bufB[bZ][rw, pl.ds(k * 16, 16)]
                            bufA[bZ][rw, pl.ds(k * 16, 16)] = _relu(a + b)
                        return rc
                    lax.fori_loop(0, C, _rows, 0)
                    pltpu.async_copy(bufA[bZ], acc.at[didx[bZ]], semS[bZ],
                                     add=True)
            return carry
        gmax = (ntrip + 2 * NBUF - 1) // NBUF
        lax.fori_loop(0, gmax, _slots, 0)
        plsc.subcore_barrier()

        pltpu.sync_copy(acc.at[pl.ds(zbase, ZROWS)],
                        agg_hbm.at[c, pl.ds(zbase, ZROWS), p])
        plsc.subcore_barrier()


def kernel(x, W_embed, b_embed, Wm0, bm0, Wu0, bu0, Wm1, bm1, Wu1, bu1,
           W_out, b_out, edge_index):
    src = edge_index[0]
    dst = edge_index[1]
    be = b_embed.reshape(1, H)
    bm0r = bm0.reshape(1, H)
    bm1r = bm1.reshape(1, H)
    bu0r = bu0.reshape(1, H)
    bu1r = bu1.reshape(1, H)
    bor = b_out.reshape(1, LAT)

    h0, hs0, hd0 = _embed_call(x, W_embed, be, Wm0[:H], Wm0[H:], bm0r)
    agg0 = _edge_kernel(hs0.reshape(CP * PAD, CW), hd0.reshape(CP * PAD, CW),
                        src, dst)
    h1, hs1, hd1 = _upd_call(h0, agg0.reshape(2, PAD, H), Wu0, bu0r,
                             Wm1[:H], Wm1[H:], bm1r)
    agg1 = _edge_kernel(hs1.reshape(CP * PAD, CW), hd1.reshape(CP * PAD, CW),
                        src, dst)
    z = _fin_call(h1, agg1.reshape(2, PAD, H), Wu1, bu1r, W_out, bor)
    return z


# speedup vs baseline: 4.8852x; 1.0335x over previous
import functools

import jax
import jax.numpy as jnp
from jax import lax
from jax.experimental import pallas as pl
from jax.experimental.pallas import tpu as pltpu
from jax.experimental.pallas import tpu_sc as plsc

N = 50000
E = 800000
ATOM = 10
H = 128
LAT = 128

NC = 2
NS = 16
NW = NC * NS

C = 64
NCHUNK = E // C
NBUF = 6
CP = 4
CW = H // CP

BN = 1024
NBLK = 49
PAD = NBLK * BN
ZROWS = PAD // NS


def _relu(v):
    return jnp.maximum(v, 0.0)


def _dot(a, b):
    return jnp.dot(a, b, preferred_element_type=jnp.float32)


def _embed_body(x_ref, We, be, WA, WB, bm, h_ref, hs_ref, hd_ref):
    h = _dot(x_ref[...], We[...]) + be[...]
    h_ref[...] = h
    hs_ref[...] = _dot(h, WA[...])
    hd_ref[...] = _dot(h, WB[...]) + bm[...]


def _upd_body(h_ref, p_ref, Wu, bu, WA, WB, bm, h_out, hs_ref, hd_ref):
    agg = p_ref[0] + p_ref[1]
    h = _relu(h_ref[...] + _dot(agg, Wu[...]) + bu[...])
    h_out[...] = h
    hs_ref[...] = _dot(h, WA[...])
    hd_ref[...] = _dot(h, WB[...]) + bm[...]


def _fin_body(h_ref, p_ref, Wu, bu, Wo, bo, z_ref):
    agg = p_ref[0] + p_ref[1]
    h = _relu(h_ref[...] + _dot(agg, Wu[...]) + bu[...])
    z_ref[...] = _dot(h, Wo[...]) + bo[...]


def _row_spec():
    return pl.BlockSpec((BN, H), lambda i: (i, 0))


def _part_spec():
    return pl.BlockSpec((2, BN, H), lambda i: (0, i, 0))


def _full(shape):
    return pl.BlockSpec(shape, lambda i: (0,) * len(shape))


_OUTH = [jax.ShapeDtypeStruct((N, H), jnp.float32),
         jax.ShapeDtypeStruct((PAD, H), jnp.float32),
         jax.ShapeDtypeStruct((PAD, H), jnp.float32)]

_embed_call = pl.pallas_call(
    _embed_body,
    grid=(NBLK,),
    in_specs=[
        pl.BlockSpec((BN, ATOM), lambda i: (i, 0)),
        _full((ATOM, H)), _full((1, H)), _full((H, H)), _full((H, H)),
        _full((1, H)),
    ],
    out_specs=[_row_spec()] * 3,
    out_shape=_OUTH,
)

_upd_call = pl.pallas_call(
    _upd_body,
    grid=(NBLK,),
    in_specs=[
        _row_spec(), _part_spec(),
        _full((H, H)), _full((1, H)), _full((H, H)), _full((H, H)),
        _full((1, H)),
    ],
    out_specs=[_row_spec()] * 3,
    out_shape=_OUTH,
)

_fin_call = pl.pallas_call(
    _fin_body,
    grid=(NBLK,),
    in_specs=[
        _row_spec(), _part_spec(),
        _full((H, H)), _full((1, H)), _full((H, LAT)), _full((1, LAT)),
    ],
    out_specs=_row_spec(),
    out_shape=jax.ShapeDtypeStruct((N, LAT), jnp.float32),
)


_sc_mesh = plsc.VectorSubcoreMesh(core_axis_name="c", subcore_axis_name="s")


@functools.partial(
    pl.kernel,
    out_type=jax.ShapeDtypeStruct((2, PAD, CP, CW), jnp.float32),
    mesh=_sc_mesh,
    compiler_params=pltpu.CompilerParams(use_tc_tiling_on_sc=False),
    scratch_types=[
        [pltpu.VMEM((C,), jnp.int32) for _ in range(NBUF)],
        [pltpu.VMEM((C,), jnp.int32) for _ in range(NBUF)],
        [pltpu.VMEM((C,), jnp.int32) for _ in range(NBUF)],
        [pltpu.VMEM((C, CW), jnp.float32) for _ in range(NBUF)],
        [pltpu.VMEM((C, CW), jnp.float32) for _ in range(NBUF)],
        pltpu.VMEM_SHARED((PAD, CW), jnp.float32),
        [pltpu.SemaphoreType.DMA for _ in range(NBUF)],
        [pltpu.SemaphoreType.DMA for _ in range(NBUF)],
        [pltpu.SemaphoreType.DMA for _ in range(NBUF)],
        [pltpu.SemaphoreType.DMA for _ in range(NBUF)],
    ],
)
def _edge_kernel(hs_hbm, hd_hbm, src_hbm, dst_hbm, agg_hbm,
                 sidx, didx, dgix, bufA, bufB, acc, semI, semA, semB, semS):
    c = lax.axis_index("c")
    s = lax.axis_index("s")
    wid = c * NS + s
    base_trips = NCHUNK // NW
    ntrip = jnp.where(wid < NCHUNK - base_trips * NW, base_trips + 1,
                      base_trips)

    for p in range(CP):
        def _zero_buf(r, carry):
            for k in range(CW // 16):
                bufA[0][r, pl.ds(k * 16, 16)] = jnp.zeros((16,), jnp.float32)
            return carry
        lax.fori_loop(0, C, _zero_buf, 0)
        zbase = s * ZROWS
        for t in range(ZROWS // C):
            pltpu.sync_copy(bufA[0], acc.at[pl.ds(zbase + t * C, C)])
        rem = ZROWS - (ZROWS // C) * C
        if rem:
            pltpu.sync_copy(bufA[0].at[pl.ds(0, rem)],
                            acc.at[pl.ds(zbase + (ZROWS // C) * C, rem)])
        plsc.subcore_barrier()

        def _slots(g, carry, p=p):
            for bb in range(NBUF):
                j = g * NBUF + bb
                bY = (bb - 1) % NBUF
                bZ = (bb - 3) % NBUF

                @pl.when((j >= NBUF) & (j - NBUF < ntrip))
                def _drain(bb=bb):
                    pltpu.make_async_copy(hs_hbm.at[pl.ds(0, C)], bufA[bb],
                                          semS[bb]).wait()

                @pl.when(j < ntrip)
                def _fire_idx(j=j, bb=bb):
                    eb = (wid + j * NW) * C
                    pltpu.async_copy(src_hbm.at[pl.ds(eb, C)], sidx[bb],
                                     semI[bb])
                    pltpu.async_copy(dst_hbm.at[pl.ds(eb, C)], didx[bb],
                                     semI[bb])

                @pl.when((j >= 1) & (j - 1 < ntrip))
                def _fire_gather(bY=bY, p=p):
                    pltpu.make_async_copy(src_hbm.at[pl.ds(0, C)],
                                          sidx[bY], semI[bY]).wait()
                    pltpu.make_async_copy(src_hbm.at[pl.ds(0, C)],
                                          didx[bY], semI[bY]).wait()
                    for k in range(C // 16):
                        sl = pl.ds(k * 16, 16)
                        sidx[bY][sl] = sidx[bY][sl] * CP + p
                        dgix[bY][sl] = didx[bY][sl] * CP + p
                    pltpu.async_copy(hs_hbm.at[sidx[bY]], bufA[bY], semA[bY])
                    pltpu.async_copy(hd_hbm.at[dgix[bY]], bufB[bY], semB[bY])

                @pl.when((j >= 3) & (j - 3 < ntrip))
                def _compute(bZ=bZ):
                    pltpu.make_async_copy(hs_hbm.at[pl.ds(0, C)], bufA[bZ],
                                          semA[bZ]).wait()
                    pltpu.make_async_copy(hs_hbm.at[pl.ds(0, C)], bufB[bZ],
                                          semB[bZ]).wait()

                    def _rows(rw, rc):
                        for k in range(CW // 16):
                            a = bufA[bZ][rw, pl.ds(k * 16, 16)]
                            b = bufB[bZ][rw, pl.ds(k * 16, 16)]
                            bufA[bZ][rw, pl.ds(k * 16, 16)] = _relu(a + b)
                        return rc
                    lax.fori_loop(0, C, _rows, 0)
                    pltpu.async_copy(bufA[bZ], acc.at[didx[bZ]], semS[bZ],
                                     add=True)
            return carry
        gmax = (ntrip + 2 * NBUF - 1) // NBUF
        lax.fori_loop(0, gmax, _slots, 0)
        plsc.subcore_barrier()

        pltpu.sync_copy(acc.at[pl.ds(zbase, ZROWS)],
                        agg_hbm.at[c, pl.ds(zbase, ZROWS), p])
        plsc.subcore_barrier()


def kernel(x, W_embed, b_embed, Wm0, bm0, Wu0, bu0, Wm1, bm1, Wu1, bu1,
           W_out, b_out, edge_index):
    src = edge_index[0]
    dst = edge_index[1]
    be = b_embed.reshape(1, H)
    bm0r = bm0.reshape(1, H)
    bm1r = bm1.reshape(1, H)
    bu0r = bu0.reshape(1, H)
    bu1r = bu1.reshape(1, H)
    bor = b_out.reshape(1, LAT)

    h0, hs0, hd0 = _embed_call(x, W_embed, be, Wm0[:H], Wm0[H:], bm0r)
    agg0 = _edge_kernel(hs0.reshape(CP * PAD, CW), hd0.reshape(CP * PAD, CW),
                        src, dst)
    h1, hs1, hd1 = _upd_call(h0, agg0.reshape(2, PAD, H), Wu0, bu0r,
                             Wm1[:H], Wm1[H:], bm1r)
    agg1 = _edge_kernel(hs1.reshape(CP * PAD, CW), hd1.reshape(CP * PAD, CW),
                        src, dst)
    z = _fin_call(h1, agg1.reshape(2, PAD, H), Wu1, bu1r, W_out, bor)
    return z
